# Initial kernel scaffold; baseline (speedup 1.0000x reference)
import jax
import jax.numpy as jnp
from jax.experimental import pallas as pl


def kernel(indices, embedding_weight):
    raise NotImplementedError("write your pallas kernel here")


# speedup vs baseline: 1.0597x; 1.0597x over previous
import jax
import jax.numpy as jnp
from jax import lax
from jax.experimental import pallas as pl
from jax.experimental.pallas import tpu as pltpu
from jax.experimental.pallas import tpu_sc as plsc

_D = 32
_NW = 32
_CHUNK = 1024
_G = 128
_LANES = 16


def _emb_lookup_cast(idx_flat, table):
    b = idx_flat.shape[0]
    b_per_w = b // _NW
    n_chunk = b_per_w // _CHUNK
    mesh = plsc.VectorSubcoreMesh(core_axis_name="c", subcore_axis_name="s")

    @pl.kernel(
        mesh=mesh,
        out_type=jax.ShapeDtypeStruct((b, 2, _LANES), jnp.bfloat16),
        scratch_types=[
            pltpu.VMEM((b_per_w,), jnp.int32),
            pltpu.VMEM((_CHUNK, _D), jnp.float32),
            pltpu.VMEM((_CHUNK, 2, _LANES), jnp.bfloat16),
            pltpu.SemaphoreType.DMA,
        ],
        compiler_params=pltpu.CompilerParams(use_tc_tiling_on_sc=False),
    )
    def body(idx_hbm, table_hbm, out_hbm, idx_v, rows_v, out_v, sem):
        wid = lax.axis_index("c") * 16 + lax.axis_index("s")
        base = wid * b_per_w
        pltpu.sync_copy(idx_hbm.at[pl.ds(base, b_per_w)], idx_v)

        for c in range(n_chunk):
            copies = []
            for g in range(_CHUNK // _G):
                r0 = c * _CHUNK + g * _G
                copies.append(
                    pltpu.async_copy(
                        table_hbm.at[idx_v.at[pl.ds(r0, _G)]],
                        rows_v.at[pl.ds(g * _G, _G)],
                        sem,
                    )
                )
            for cp in copies:
                cp.wait()

            def cast_row(i):
                a = rows_v[i, pl.ds(0, _LANES)].astype(jnp.bfloat16)
                bb = rows_v[i, pl.ds(_LANES, _LANES)].astype(jnp.bfloat16)
                out_v[i, :, :] = jnp.stack([a, bb])

            pl.loop(0, _CHUNK, unroll=4)(cast_row)

            pltpu.sync_copy(
                out_v, out_hbm.at[pl.ds(base + c * _CHUNK, _CHUNK)]
            )

    return body(idx_flat, table)


def kernel(indices, embedding_weight):
    idx_flat = indices.reshape(-1).astype(jnp.int32)
    out = _emb_lookup_cast(idx_flat, embedding_weight)
    return out.reshape(*indices.shape, _D)


# speedup vs baseline: 1.1590x; 1.0936x over previous
import jax
import jax.numpy as jnp
from jax import lax
from jax.experimental import pallas as pl
from jax.experimental.pallas import tpu as pltpu
from jax.experimental.pallas import tpu_sc as plsc

_D = 32
_NW = 32
_CHUNK = 1024
_G = 128
_LANES = 16


def _emb_lookup_cast(idx_flat, table):
    b = idx_flat.shape[0]
    b_per_w = b // _NW
    n_chunk = b_per_w // _CHUNK
    mesh = plsc.VectorSubcoreMesh(core_axis_name="c", subcore_axis_name="s")

    @pl.kernel(
        mesh=mesh,
        out_type=jax.ShapeDtypeStruct((b, 2, _LANES), jnp.bfloat16),
        scratch_types=[
            pltpu.VMEM((b_per_w,), jnp.int32),
            pltpu.VMEM((2, _CHUNK, _D), jnp.float32),
            pltpu.VMEM((2, _CHUNK, 2, _LANES), jnp.bfloat16),
            pltpu.SemaphoreType.DMA,
            pltpu.SemaphoreType.DMA,
            pltpu.SemaphoreType.DMA,
            pltpu.SemaphoreType.DMA,
        ],
        compiler_params=pltpu.CompilerParams(use_tc_tiling_on_sc=False),
    )
    def body(idx_hbm, table_hbm, out_hbm, idx_v, rows_v, out_v, g0, g1, o0, o1):
        gsem = (g0, g1)
        osem = (o0, o1)
        wid = lax.axis_index("c") * 16 + lax.axis_index("s")
        base = wid * b_per_w
        pltpu.sync_copy(idx_hbm.at[pl.ds(base, b_per_w)], idx_v)

        def fire_gathers(c, buf):
            cps = []
            for g in range(_CHUNK // _G):
                r0 = c * _CHUNK + g * _G
                cps.append(
                    pltpu.async_copy(
                        table_hbm.at[idx_v.at[pl.ds(r0, _G)]],
                        rows_v.at[buf, pl.ds(g * _G, _G)],
                        gsem[buf],
                    )
                )
            return cps

        pend_g = {0: fire_gathers(0, 0)}
        pend_o = {}
        for c in range(n_chunk):
            buf = c % 2
            if c + 1 < n_chunk:
                pend_g[c + 1] = fire_gathers(c + 1, 1 - buf)
            for cp in pend_g.pop(c):
                cp.wait()
            if c - 2 in pend_o:
                pend_o.pop(c - 2).wait()

            def cast_row(i, buf=buf):
                a = rows_v[buf, i, pl.ds(0, _LANES)].astype(jnp.bfloat16)
                bb = rows_v[buf, i, pl.ds(_LANES, _LANES)].astype(jnp.bfloat16)
                out_v[buf, i, :, :] = jnp.stack([a, bb])

            plsc.parallel_loop(0, _CHUNK, unroll=4)(cast_row)

            pend_o[c] = pltpu.async_copy(
                out_v.at[buf],
                out_hbm.at[pl.ds(base + c * _CHUNK, _CHUNK)],
                osem[buf],
            )
        for c in sorted(pend_o):
            pend_o[c].wait()

    return body(idx_flat, table)


def kernel(indices, embedding_weight):
    idx_flat = indices.reshape(-1).astype(jnp.int32)
    out = _emb_lookup_cast(idx_flat, embedding_weight)
    return out.reshape(*indices.shape, _D)


# speedup vs baseline: 2.4657x; 2.1275x over previous
import jax
import jax.numpy as jnp
from jax import lax
from jax.experimental import pallas as pl
from jax.experimental.pallas import tpu as pltpu
from jax.experimental.pallas import tpu_sc as plsc

_V = 1_000_000
_D = 32
_C = 26
_B = 16384
_NS = 16
_BT = _B // _NS
_STRIDE = 1 << 20
_BR = 512
_VCH = _BR * 128
_NVB = _STRIDE // _VCH
_LANES = 16


def _pack_tc(w_t):
    def body(in_ref, out_ref):
        for p in range(4):
            a = in_ref[2 * p, :].reshape(_BR, 128)
            b = in_ref[2 * p + 1, :].reshape(_BR, 128)
            a16 = jax.lax.bitcast_convert_type(
                a.astype(jnp.bfloat16), jnp.uint16)
            b16 = jax.lax.bitcast_convert_type(
                b.astype(jnp.bfloat16), jnp.uint16)
            w = a16.astype(jnp.uint32) | (b16.astype(jnp.uint32) << 16)
            out_ref[p, :, :] = w.astype(jnp.int32)

    return pl.pallas_call(
        body,
        grid=(4, _NVB),
        in_specs=[pl.BlockSpec((8, _VCH), lambda pg, vb: (pg, vb))],
        out_specs=pl.BlockSpec((4, _BR, 128), lambda pg, vb: (pg, vb, 0)),
        out_shape=jax.ShapeDtypeStruct((16, _STRIDE // 128, 128), jnp.int32),
    )(w_t)


def _gather_sc(idx_t, wp_flat):
    mesh = plsc.VectorSubcoreMesh(core_axis_name="c", subcore_axis_name="s")

    @pl.kernel(
        mesh=mesh,
        out_type=jax.ShapeDtypeStruct((_C, 4, 128, 4, 128), jnp.int32),
        scratch_types=[
            pltpu.VMEM((_C, _BT), jnp.int32),
            pltpu.VMEM((_C, _BT), jnp.int32),
            pltpu.VMEM_SHARED((_STRIDE,), jnp.int32),
            pltpu.SemaphoreType.DMA,
            pltpu.SemaphoreType.DMA,
            pltpu.SemaphoreType.DMA,
        ],
        compiler_params=pltpu.CompilerParams(use_tc_tiling_on_sc=False),
    )
    def body(idx_hbm, wp_hbm, out_hbm, idx_v, gb, col_s, ssem, gsem, osem):
        core = lax.axis_index("c")
        s = lax.axis_index("s")

        pltpu.sync_copy(
            idx_hbm.at[pl.ds(0, _C), pl.ds(s * _BT, _BT)], idx_v
        )

        def drain_g():
            pltpu.make_async_copy(
                idx_hbm.at[pl.ds(0, _C), pl.ds(0, _BT)], gb, gsem
            ).wait()

        def drain_o():
            pltpu.make_async_copy(
                idx_hbm.at[pl.ds(0, _C), pl.ds(0, _BT)], gb, osem
            ).wait()

        for q in range(8):
            iq = q // 4
            srq = q % 4
            pr = core * 8 + q
            ig = core * 2 + iq

            sl = _STRIDE // _NS
            pltpu.async_copy(
                wp_hbm.at[pl.ds(pr * _STRIDE + s * sl, sl)],
                col_s.at[pl.ds(s * sl, sl)],
                ssem,
            )
            pltpu.make_async_copy(
                wp_hbm.at[pl.ds(0, sl)], col_s.at[pl.ds(0, sl)], ssem
            ).wait()
            plsc.subcore_barrier()

            if q >= 1:
                drain_o()

            def fire_c(c):
                def fire_g(jl):
                    pltpu.async_copy(
                        col_s.at[idx_v.at[c, pl.ds(jl * 128, 128)]],
                        gb.at[c, pl.ds(jl * 128, 128)],
                        gsem,
                    )

                pl.loop(0, 8)(fire_g)

            pl.loop(0, _C)(fire_c)
            drain_g()

            def out_fire(c):
                def one(jl):
                    pltpu.async_copy(
                        gb.at[c, pl.ds(jl * 128, 128)],
                        out_hbm.at[c, ig, 8 * s + jl, srq],
                        osem,
                    )

                pl.loop(0, 8)(one)

            pl.loop(0, _C)(out_fire)
            plsc.subcore_barrier()

        drain_o()

    return body(idx_t, wp_flat)


def kernel(indices, embedding_weight):
    idx_t = jnp.transpose(indices).astype(jnp.int32)
    w_t = jnp.transpose(embedding_weight)
    wp = _pack_tc(w_t)
    out5 = _gather_sc(idx_t, wp.reshape(16 * _STRIDE))
    out6 = jax.lax.bitcast_convert_type(out5, jnp.bfloat16)
    return out6.transpose(2, 4, 0, 1, 3, 5).reshape(_B, _C, _D)
